# baseline (device time: 43176 ns/iter reference)
import jax
import jax.numpy as jnp
from jax import lax
from jax.experimental import pallas as pl
from jax.experimental.pallas import tpu as pltpu

N_DEV = 8
EPS = 1e-5


def kernel(x, gamma, beta):
    m, n_per = x.shape
    n_total = n_per * N_DEV

    def body(x_ref, g_ref, b_ref, out_ref, mine_ref, comm_ref, send_sems, recv_sems):
        my = lax.axis_index("i")

        barrier = pltpu.get_barrier_semaphore()
        for d in range(1, N_DEV):
            peer = (my + d) % N_DEV
            pl.semaphore_signal(
                barrier, inc=1,
                device_id=(peer,), device_id_type=pl.DeviceIdType.MESH,
            )
        pl.semaphore_wait(barrier, N_DEV - 1)

        xf = x_ref[:, :].astype(jnp.float32)
        s = jnp.sum(xf, axis=1, keepdims=True)
        sq = jnp.sum(xf * xf, axis=1, keepdims=True)
        mine_ref[:, :] = jnp.concatenate([s, sq], axis=1)

        sends = []
        for d in range(1, N_DEV):
            peer = (my + d) % N_DEV
            slot = N_DEV - 1 - d
            rdma = pltpu.make_async_remote_copy(
                src_ref=mine_ref,
                dst_ref=comm_ref.at[slot],
                send_sem=send_sems.at[d - 1],
                recv_sem=recv_sems.at[slot],
                device_id=(peer,),
                device_id_type=pl.DeviceIdType.MESH,
            )
            rdma.start()
            sends.append(rdma)

        for slot in range(N_DEV - 1):
            recv = pltpu.make_async_remote_copy(
                src_ref=mine_ref,
                dst_ref=comm_ref.at[slot],
                send_sem=send_sems.at[0],
                recv_sem=recv_sems.at[slot],
                device_id=(my,),
                device_id_type=pl.DeviceIdType.MESH,
            )
            recv.wait_recv()

        total = mine_ref[:, :] + jnp.sum(comm_ref[:, :, :], axis=0)
        mean = total[:, 0:1] / n_total
        var = total[:, 1:2] / n_total - mean * mean
        rstd = lax.rsqrt(var + EPS)
        out_ref[:, :] = (g_ref[:] * ((xf - mean) * rstd) + b_ref[:]).astype(
            out_ref.dtype
        )

        for rdma in sends:
            rdma.wait_send()

    return pl.pallas_call(
        body,
        out_shape=jax.ShapeDtypeStruct((m, n_per), jnp.float32),
        in_specs=[
            pl.BlockSpec(memory_space=pltpu.VMEM),
            pl.BlockSpec(memory_space=pltpu.VMEM),
            pl.BlockSpec(memory_space=pltpu.VMEM),
        ],
        out_specs=pl.BlockSpec(memory_space=pltpu.VMEM),
        scratch_shapes=[
            pltpu.VMEM((m, 2), jnp.float32),
            pltpu.VMEM((N_DEV - 1, m, 2), jnp.float32),
            pltpu.SemaphoreType.DMA((N_DEV - 1,)),
            pltpu.SemaphoreType.DMA((N_DEV - 1,)),
        ],
        compiler_params=pltpu.CompilerParams(collective_id=0),
    )(x, gamma, beta)


# device time: 12537 ns/iter; 3.4439x vs baseline; 3.4439x over previous
import jax
import jax.numpy as jnp
from jax import lax
from jax.experimental import pallas as pl
from jax.experimental.pallas import tpu as pltpu

N_DEV = 8
EPS = 1e-5


def kernel(x, gamma, beta):
    m, n_per = x.shape
    n_total = n_per * N_DEV

    def body(x_ref, g_ref, b_ref, out_ref, mine_ref, comm_ref, send_sems, recv_sems):
        my = lax.axis_index("i")

        xf = x_ref[:, :].astype(jnp.float32)
        ones_row = jnp.ones((1, n_per), jnp.float32)
        dn = (((1,), (1,)), ((), ()))
        sT = lax.dot_general(ones_row, xf, dn, preferred_element_type=jnp.float32)
        sqT = lax.dot_general(
            ones_row, xf * xf, dn, preferred_element_type=jnp.float32
        )
        mine_ref[0:1, :] = sT
        mine_ref[1:2, :] = sqT

        barrier = pltpu.get_barrier_semaphore()
        for d in range(1, N_DEV):
            peer = (my + d) % N_DEV
            pl.semaphore_signal(
                barrier, inc=1,
                device_id=(peer,), device_id_type=pl.DeviceIdType.MESH,
            )
        pl.semaphore_wait(barrier, N_DEV - 1)

        sends = []
        for d in range(1, N_DEV):
            peer = (my + d) % N_DEV
            slot = N_DEV - 1 - d
            rdma = pltpu.make_async_remote_copy(
                src_ref=mine_ref,
                dst_ref=comm_ref.at[slot],
                send_sem=send_sems.at[d - 1],
                recv_sem=recv_sems.at[slot],
                device_id=(peer,),
                device_id_type=pl.DeviceIdType.MESH,
            )
            rdma.start()
            sends.append(rdma)

        for slot in range(N_DEV - 1):
            recv = pltpu.make_async_remote_copy(
                src_ref=mine_ref,
                dst_ref=comm_ref.at[slot],
                send_sem=send_sems.at[0],
                recv_sem=recv_sems.at[slot],
                device_id=(my,),
                device_id_type=pl.DeviceIdType.MESH,
            )
            recv.wait_recv()

        total = (mine_ref[:, :] + jnp.sum(comm_ref[:, :, :], axis=0)).T
        mean = total[:, 0:1] / n_total
        var = total[:, 1:2] / n_total - mean * mean
        rstd = lax.rsqrt(var + EPS)
        out_ref[:, :] = (g_ref[:] * ((xf - mean) * rstd) + b_ref[:]).astype(
            out_ref.dtype
        )

        for rdma in sends:
            rdma.wait_send()

    return pl.pallas_call(
        body,
        out_shape=jax.ShapeDtypeStruct((m, n_per), jnp.float32),
        in_specs=[
            pl.BlockSpec(memory_space=pltpu.VMEM),
            pl.BlockSpec(memory_space=pltpu.VMEM),
            pl.BlockSpec(memory_space=pltpu.VMEM),
        ],
        out_specs=pl.BlockSpec(memory_space=pltpu.VMEM),
        scratch_shapes=[
            pltpu.VMEM((2, m), jnp.float32),
            pltpu.VMEM((N_DEV - 1, 2, m), jnp.float32),
            pltpu.SemaphoreType.DMA((N_DEV - 1,)),
            pltpu.SemaphoreType.DMA((N_DEV - 1,)),
        ],
        compiler_params=pltpu.CompilerParams(collective_id=0),
    )(x, gamma, beta)
